# Initial kernel scaffold; baseline (speedup 1.0000x reference)
import jax
import jax.numpy as jnp
from jax.experimental import pallas as pl


def kernel(user_indices, item_indices, user_emb, item_emb, user_content_emb, item_content_emb, W1, b1, W2, b2):
    raise NotImplementedError("write your pallas kernel here")


# speedup vs baseline: 2.4172x; 2.4172x over previous
import functools

import jax
import jax.numpy as jnp
from jax import lax
from jax.experimental import pallas as pl
from jax.experimental.pallas import tpu as pltpu
from jax.experimental.pallas import tpu_sc as plsc

EMBED = 128
CHUNK = 128


def _make_sc_gather(batch, num_rows_u, num_rows_i, dtype):
    info = plsc.get_sparse_core_info()
    nc, ns = info.num_cores, info.num_subcores
    nw = nc * ns
    assert batch % (nw * CHUNK) == 0, (batch, nw, CHUNK)
    kpw = batch // (nw * CHUNK)
    mesh = plsc.VectorSubcoreMesh(core_axis_name="c", subcore_axis_name="s")

    out_t = tuple(
        jax.ShapeDtypeStruct((batch, EMBED), dtype) for _ in range(4)
    )

    @functools.partial(
        pl.kernel,
        out_type=out_t,
        mesh=mesh,
        scratch_types=[
            pltpu.VMEM((kpw, CHUNK), jnp.int32),
            pltpu.VMEM((kpw, CHUNK), jnp.int32),
            pltpu.VMEM((CHUNK, EMBED), dtype),
            pltpu.VMEM((CHUNK, EMBED), dtype),
            pltpu.SemaphoreType.DMA,
            pltpu.SemaphoreType.DMA,
        ],
    )
    def sc_gather(uidx_hbm, iidx_hbm, ue_hbm, ie_hbm, uc_hbm, ic_hbm,
                  out_ue, out_ie, out_uc, out_ic,
                  idx_u, idx_i, rows_a, rows_b, sem_a, sem_b):
        wid = lax.axis_index("s") * nc + lax.axis_index("c")
        base = wid * kpw
        pltpu.sync_copy(uidx_hbm.at[pl.ds(base, kpw)], idx_u)
        pltpu.sync_copy(iidx_hbm.at[pl.ds(base, kpw)], idx_i)

        work = []
        for table, idxv, out in ((ue_hbm, idx_u, out_ue),
                                 (ie_hbm, idx_i, out_ie),
                                 (uc_hbm, idx_u, out_uc),
                                 (ic_hbm, idx_i, out_ic)):
            for j in range(kpw):
                work.append((table, idxv, j, out))

        bufs = (rows_a, rows_b)
        sems = (sem_a, sem_b)
        for k, (table, idxv, j, out) in enumerate(work):
            buf, sem = bufs[k % 2], sems[k % 2]
            pltpu.async_copy(table.at[idxv.at[j]], buf, sem).wait()
            pltpu.sync_copy(buf, out.at[pl.ds((base + j) * CHUNK, CHUNK)])

    return sc_gather


def _tc_body(ue_ref, ie_ref, uc_ref, ic_ref, w1a_ref, w1b_ref, b1_ref,
             w2_ref, b2_ref, out_ref):
    cf = jnp.sum(ue_ref[...] * ie_ref[...], axis=1, keepdims=True)
    h = jnp.dot(uc_ref[...], w1a_ref[...], preferred_element_type=jnp.float32)
    h = h + jnp.dot(ic_ref[...], w1b_ref[...],
                    preferred_element_type=jnp.float32)
    h = jnp.maximum(h + b1_ref[...], 0.0)
    out = jnp.dot(h, w2_ref[...], preferred_element_type=jnp.float32)
    out_ref[...] = cf + out + b2_ref[...]


def _tc_mlp(ue, ie, uc, ic, w1a, w1b, b1, w2, b2, blk):
    batch = ue.shape[0]
    hid = w2.shape[0]
    grid = (batch // blk,)
    row_spec = pl.BlockSpec((blk, EMBED), lambda i: (i, 0))
    full = lambda shape: pl.BlockSpec(shape, lambda i: (0, 0))
    return pl.pallas_call(
        _tc_body,
        grid=grid,
        in_specs=[
            row_spec, row_spec, row_spec, row_spec,
            full((EMBED, hid)), full((EMBED, hid)), full((1, hid)),
            full((hid, EMBED)), full((1, EMBED)),
        ],
        out_specs=row_spec,
        out_shape=jax.ShapeDtypeStruct((batch, EMBED), jnp.float32),
    )(ue, ie, uc, ic, w1a, w1b, b1, w2, b2)


def kernel(user_indices, item_indices, user_emb, item_emb,
           user_content_emb, item_content_emb, W1, b1, W2, b2):
    batch = user_indices.shape[0]
    uidx2 = user_indices.astype(jnp.int32).reshape(batch // CHUNK, CHUNK)
    iidx2 = item_indices.astype(jnp.int32).reshape(batch // CHUNK, CHUNK)

    sc_gather = _make_sc_gather(batch, user_emb.shape[0], item_emb.shape[0],
                                user_emb.dtype)
    ue_g, ie_g, uc_g, ic_g = sc_gather(
        uidx2, iidx2, user_emb, item_emb, user_content_emb, item_content_emb)

    hid = W2.shape[0]
    w1a, w1b = W1[:EMBED], W1[EMBED:]
    return _tc_mlp(ue_g, ie_g, uc_g, ic_g, w1a, w1b,
                   b1.reshape(1, hid), W2, b2.reshape(1, EMBED), blk=2048)


# speedup vs baseline: 2.8328x; 1.1719x over previous
import functools

import jax
import jax.numpy as jnp
from jax import lax
from jax.experimental import pallas as pl
from jax.experimental.pallas import tpu as pltpu
from jax.experimental.pallas import tpu_sc as plsc

EMBED = 128
CHUNK = 128


STREAM = 1


def _make_sc_gather(batch, dtype):
    info = plsc.get_sparse_core_info()
    nc, ns = info.num_cores, info.num_subcores
    nw = nc * ns
    assert batch % (nw * CHUNK * STREAM) == 0, (batch, nw)
    kpw = batch // (nw * CHUNK)
    spw = kpw // STREAM
    nchunks = batch // CHUNK
    mesh = plsc.VectorSubcoreMesh(core_axis_name="c", subcore_axis_name="s")

    out_t = tuple(
        jax.ShapeDtypeStruct((nchunks, CHUNK, EMBED), dtype) for _ in range(4)
    )

    @functools.partial(
        pl.kernel,
        out_type=out_t,
        mesh=mesh,
        scratch_types=[
            pltpu.VMEM((kpw, CHUNK), jnp.int32),
            pltpu.VMEM((kpw, CHUNK), jnp.int32),
            pltpu.VMEM((CHUNK, EMBED), dtype),
            pltpu.VMEM((CHUNK, EMBED), dtype),
            pltpu.SemaphoreType.DMA,
            pltpu.SemaphoreType.DMA,
            pltpu.SemaphoreType.DMA,
            pltpu.SemaphoreType.DMA,
        ],
    )
    def sc_gather(uidx_hbm, iidx_hbm, ue_hbm, ie_hbm, uc_hbm, ic_hbm,
                  out_ue, out_ie, out_uc, out_ic,
                  idx_u, idx_i, rows_a, rows_b,
                  gsem_a, gsem_b, ssem_a, ssem_b):
        wid = lax.axis_index("s") * nc + lax.axis_index("c")
        base = wid * kpw
        pltpu.sync_copy(uidx_hbm.at[pl.ds(base, kpw)], idx_u)
        pltpu.sync_copy(iidx_hbm.at[pl.ds(base, kpw)], idx_i)

        work = []
        for table, idxv, out in ((ue_hbm, idx_u, out_ue),
                                 (ie_hbm, idx_i, out_ie),
                                 (uc_hbm, idx_u, out_uc),
                                 (ic_hbm, idx_i, out_ic)):
            for j in range(spw):
                work.append((table, idxv, j, out))

        bufs = (rows_a, rows_b)
        gsems = (gsem_a, gsem_b)
        ssems = (ssem_a, ssem_b)
        n = len(work)

        def start_gather(k):
            table, idxv, j, _ = work[k]
            b = k % 2
            return pltpu.async_copy(table.at[idxv.at[j]], bufs[b], gsems[b])

        def start_store(k):
            _, _, j, out = work[k]
            b = k % 2
            return pltpu.async_copy(bufs[b], out.at[base + j], ssems[b])

        store_cp = [None, None]
        gather_cp = [None, None]
        gather_cp[0] = start_gather(0)
        for k in range(n):
            b = k % 2
            nb = (k + 1) % 2
            if k + 1 < n:
                if store_cp[nb] is not None:
                    store_cp[nb].wait()
                gather_cp[nb] = start_gather(k + 1)
            gather_cp[b].wait()
            store_cp[b] = start_store(k)
        for b in range(2):
            if store_cp[b] is not None:
                store_cp[b].wait()

    return sc_gather


def _tc_body(ue_ref, ie_ref, uc_ref, ic_ref, w1a_ref, w1b_ref, b1_ref,
             w2_ref, b2_ref, out_ref):
    cf = jnp.sum(ue_ref[...] * ie_ref[...], axis=1, keepdims=True)
    h = jnp.dot(uc_ref[...], w1a_ref[...], preferred_element_type=jnp.float32)
    h = h + jnp.dot(ic_ref[...], w1b_ref[...],
                    preferred_element_type=jnp.float32)
    h = jnp.maximum(h + b1_ref[...], 0.0)
    out = jnp.dot(h, w2_ref[...], preferred_element_type=jnp.float32)
    out_ref[...] = cf + out + b2_ref[...]


def _tc_mlp(ue, ie, uc, ic, w1a, w1b, b1, w2, b2, blk):
    batch = ue.shape[0]
    hid = w2.shape[0]
    grid = (batch // blk,)
    row_spec = pl.BlockSpec((blk, EMBED), lambda i: (i, 0))
    full = lambda shape: pl.BlockSpec(shape, lambda i: (0, 0))
    return pl.pallas_call(
        _tc_body,
        grid=grid,
        in_specs=[
            row_spec, row_spec, row_spec, row_spec,
            full((EMBED, hid)), full((EMBED, hid)), full((1, hid)),
            full((hid, EMBED)), full((1, EMBED)),
        ],
        out_specs=row_spec,
        out_shape=jax.ShapeDtypeStruct((batch, EMBED), jnp.float32),
    )(ue, ie, uc, ic, w1a, w1b, b1, w2, b2)


def kernel(user_indices, item_indices, user_emb, item_emb,
           user_content_emb, item_content_emb, W1, b1, W2, b2):
    batch = user_indices.shape[0]
    uidx2 = user_indices.astype(jnp.int32).reshape(batch // CHUNK, CHUNK)
    iidx2 = item_indices.astype(jnp.int32).reshape(batch // CHUNK, CHUNK)

    sc_gather = _make_sc_gather(batch, user_emb.dtype)
    ue_g, ie_g, uc_g, ic_g = (
        x.reshape(batch, EMBED) for x in sc_gather(
            uidx2, iidx2, user_emb, item_emb,
            user_content_emb, item_content_emb))

    hid = W2.shape[0]
    w1a, w1b = W1[:EMBED], W1[EMBED:]
    return _tc_mlp(ue_g, ie_g, uc_g, ic_g, w1a, w1b,
                   b1.reshape(1, hid), W2, b2.reshape(1, EMBED), blk=2048)
